# baseline (speedup 1.0000x reference)
import functools
import jax
import jax.numpy as jnp
from jax import lax
from jax.experimental import pallas as pl
from jax.experimental.pallas import tpu as pltpu
from jax.experimental.pallas import tpu_sc as plsc

_B, _L, _D = 1024, 200, 128
_NV, _NCAT, _H = 64, 512, 16
_N = _B * _L
_T = 6400
_GRID = _N // _T
_NBUF = 4

_NC, _NS, _LN = 2, 16, 16
_NW = _NC * _NS
_CHTOK = _N // _NW
_VPT = _CHTOK // _LN
_CH = 128
_MAXCH = _CHTOK // _CH
_PAD = _CHTOK + _CH


def _tc_block(vid, val, P, W2c, Jn, gam, bet):
    lane = jax.lax.broadcasted_iota(jnp.int32, (_T, _D), 1)
    oh2 = (vid == lane % _NV).astype(jnp.float32)
    z = oh2 * jnp.where(lane < _NV, val, 1.0)
    hp = jnp.dot(z, P, preferred_element_type=jnp.float32)
    h = jnp.maximum(hp, 0.0)
    c = jnp.dot(h, W2c, preferred_element_type=jnp.float32)
    vvar = jnp.dot(c * c, Jn, preferred_element_type=jnp.float32)
    return c * jax.lax.rsqrt(vvar + 1e-5) * gam + bet


def _tc_base_body(vid0_ref, val0_ref, vid1_ref, val1_ref, P_ref, W2c_ref,
                  Jn_ref, gam_ref, bet_ref, out0_ref, out1_ref):
    P = P_ref[...]
    W2c = W2c_ref[...]
    Jn = Jn_ref[...]
    gam = gam_ref[...]
    bet = bet_ref[...]
    out0_ref[...] = _tc_block(vid0_ref[...], val0_ref[...], P, W2c, Jn, gam, bet)
    out1_ref[...] = _tc_block(vid1_ref[...], val1_ref[...], P, W2c, Jn, gam, bet)


def _lnt_body(cat_ref, gam_ref, bet_ref, out_ref):
    t = cat_ref[...]
    mean = jnp.mean(t, axis=1, keepdims=True)
    var = jnp.mean((t - mean) ** 2, axis=1, keepdims=True)
    out_ref[...] = (t - mean) / jnp.sqrt(var + 1e-5) * gam_ref[...] + bet_ref[...]


_sc_mesh = plsc.VectorSubcoreMesh(core_axis_name="c", subcore_axis_name="s",
                                  num_cores=_NC, num_subcores=_NS)


@functools.partial(
    pl.kernel,
    out_type=(),
    mesh=_sc_mesh,
    compiler_params=pltpu.CompilerParams(needs_layout_passes=False),
    scratch_types=[
        pltpu.VMEM((_CHTOK,), jnp.int32),
        pltpu.VMEM((_CHTOK,), jnp.int32),
        pltpu.VMEM((_NV,), jnp.int32),
        pltpu.VMEM((_PAD,), jnp.int32),
        pltpu.VMEM((_PAD,), jnp.int32),
        pltpu.VMEM((_MAXCH + 1, _CH), jnp.int32),
        pltpu.VMEM((4, _CH, _D), jnp.float32),
        pltpu.SemaphoreType.DMA,
        pltpu.SemaphoreType.DMA,
    ],
)
def _sc_overwrite(lnt_hbm, vid_hbm, cid_hbm, vty_hbm, out_hbm,
                  vids_v, cids_v, vty_v, srcf, dstf, dst2, rows, gsem, ssem):
    wid = lax.axis_index("s") * _NC + lax.axis_index("c")
    base = wid * _CHTOK
    pltpu.sync_copy(vid_hbm.at[pl.ds(base, _CHTOK)], vids_v)
    pltpu.sync_copy(cid_hbm.at[pl.ds(base, _CHTOK)], cids_v)
    pltpu.sync_copy(vty_hbm, vty_v)

    def step(i, ptr):
        sl = pl.ds(i * _LN, _LN)
        vid = vids_v[sl]
        cid = cids_v[sl]
        vt = plsc.load_gather(vty_v, [vid])
        m = (vt == 1) & (cid >= 0)
        tok = base + i * _LN + lax.iota(jnp.int32, _LN)
        plsc.store_compressed(srcf.at[pl.ds(ptr, _LN)], cid, mask=m)
        plsc.store_compressed(dstf.at[pl.ds(ptr, _LN)], tok, mask=m)
        cnt = plsc.all_reduce_population_count(m)
        return ptr + jnp.max(cnt)

    count = lax.fori_loop(0, _VPT, step, jnp.int32(0))

    @pl.when(count > 0)
    def _():
        lane0 = lax.iota(jnp.int32, _LN) == 0
        s0 = srcf[pl.ds(0, _LN)]
        d0 = dstf[pl.ds(0, _LN)]
        spad = jnp.full((_LN,), jnp.max(jnp.where(lane0, s0, 0)), jnp.int32)
        dpad = jnp.full((_LN,), jnp.max(jnp.where(lane0, d0, 0)), jnp.int32)
        for j in range(_CH // _LN):
            srcf[pl.ds(count + j * _LN, _LN)] = spad
            dstf[pl.ds(count + j * _LN, _LN)] = dpad
        for c in range(_MAXCH + 1):
            for k in range(_CH // _LN):
                dst2[c, pl.ds(k * _LN, _LN)] = dstf[pl.ds(c * _CH + k * _LN, _LN)]

        nch = (count + _CH - 1) // _CH
        pltpu.async_copy(lnt_hbm.at[srcf.at[pl.ds(0, _CH)]], rows.at[0], gsem)

        @pl.when(nch > 1)
        def _():
            pltpu.async_copy(lnt_hbm.at[srcf.at[pl.ds(_CH, _CH)]],
                             rows.at[1], gsem)

        def chunk(c, carry):
            cur = lax.rem(c, 4)

            @pl.when(c >= 2)
            def _():
                pltpu.make_async_copy(rows.at[cur], out_hbm.at[dst2.at[c]],
                                      ssem).wait()

            @pl.when(c + 2 < nch)
            def _():
                pltpu.async_copy(
                    lnt_hbm.at[srcf.at[pl.ds((c + 2) * _CH, _CH)]],
                    rows.at[lax.rem(c + 2, 4)], gsem)

            pltpu.make_async_copy(
                lnt_hbm.at[srcf.at[pl.ds(0, _CH)]], rows.at[cur], gsem).wait()
            pltpu.async_copy(rows.at[cur], out_hbm.at[dst2.at[c]], ssem)
            return carry

        lax.fori_loop(0, nch, chunk, jnp.int32(0))

        @pl.when(nch > 1)
        def _():
            pltpu.make_async_copy(rows.at[0], out_hbm.at[dst2.at[0]],
                                  ssem).wait()
        pltpu.make_async_copy(rows.at[0], out_hbm.at[dst2.at[0]], ssem).wait()


@jax.jit
def kernel(variate_ids, value_num, cat_ids, variate_type, numeric_means,
           numeric_stds, w1, b1, W2, b2, cat_table, ln_gamma, ln_beta):
    vid2 = variate_ids.reshape(_N, 1).astype(jnp.int32)
    val2 = value_num.reshape(_N, 1)
    tf = variate_type.astype(jnp.int32)

    isnum = (tf == 0).astype(jnp.float32)
    s = 1.0 / (numeric_stds + 1e-6)
    t = -numeric_means * s
    P = jnp.full((_D, _D), 0.0, jnp.float32)
    P = P.at[:_NV, :_H].set(s[:, None] * w1[None, :])
    P = P.at[_NV:, :_H].set(t[:, None] * w1[None, :] + b1[None, :]
                            - 1e20 * (1.0 - isnum)[:, None])
    P = P.at[_NV:, _H].set(isnum)
    P = P.at[_NV:, _H + 1:].set(-1.0)
    W2ext = jnp.zeros((_D, _D), jnp.float32)
    W2ext = W2ext.at[:_H, :].set(W2)
    W2ext = W2ext.at[_H, :].set(b2)
    W2c = W2ext - jnp.mean(W2ext, axis=1, keepdims=True)
    Jn = jnp.full((_D, _D), 1.0 / _D, jnp.float32)

    grid2 = _N // (2 * _T)
    tok = lambda i: (i, 0)
    tok_hi = lambda i: (i + grid2, 0)
    full = lambda i: (0, 0)
    b0, b1 = pl.pallas_call(
        _tc_base_body,
        grid=(grid2,),
        in_specs=[
            pl.BlockSpec((_T, 1), tok),
            pl.BlockSpec((_T, 1), tok),
            pl.BlockSpec((_T, 1), tok_hi),
            pl.BlockSpec((_T, 1), tok_hi),
            pl.BlockSpec((_D, _D), full),
            pl.BlockSpec((_D, _D), full),
            pl.BlockSpec((_D, _D), full),
            pl.BlockSpec((1, _D), full),
            pl.BlockSpec((1, _D), full),
        ],
        out_specs=[pl.BlockSpec((_T, _D), tok),
                   pl.BlockSpec((_T, _D), tok)],
        out_shape=[jax.ShapeDtypeStruct((_N // 2, _D), jnp.float32),
                   jax.ShapeDtypeStruct((_N // 2, _D), jnp.float32)],
    )(vid2, val2, vid2, val2, P, W2c, Jn, ln_gamma.reshape(1, _D),
      ln_beta.reshape(1, _D))
    base = jnp.concatenate([b0, b1], axis=0)

    lnt = pl.pallas_call(
        _lnt_body,
        in_specs=[
            pl.BlockSpec((_NCAT, _D), lambda: (0, 0)),
            pl.BlockSpec((1, _D), lambda: (0, 0)),
            pl.BlockSpec((1, _D), lambda: (0, 0)),
        ],
        out_specs=pl.BlockSpec((_NCAT, _D), lambda: (0, 0)),
        out_shape=jax.ShapeDtypeStruct((_NCAT, _D), jnp.float32),
    )(cat_table, ln_gamma.reshape(1, _D), ln_beta.reshape(1, _D))

    out_ref = jax.new_ref(base)
    _sc_overwrite(lnt, variate_ids.reshape(_N).astype(jnp.int32),
                  cat_ids.reshape(_N).astype(jnp.int32), tf, out_ref)
    return jax.freeze(out_ref).reshape(_B, _L, _D)


# speedup vs baseline: 1.1554x; 1.1554x over previous
import functools
import jax
import jax.numpy as jnp
from jax import lax
from jax.experimental import pallas as pl
from jax.experimental.pallas import tpu as pltpu
from jax.experimental.pallas import tpu_sc as plsc

_B, _L, _D = 1024, 200, 128
_NV, _NCAT, _H = 64, 512, 16
_N = _B * _L
_T = 12800

_NC, _NS, _LN = 2, 16, 16
_NW = _NC * _NS
_CHTOK = _N // _NW
_VPT = _CHTOK // _LN
_CH = 128
_MAXCH = _CHTOK // _CH
_PAD = _CHTOK + _CH


def _tc_base_body(vid_ref, val_ref, P_ref, W2c_ref, Jn_ref, gam_ref,
                  bet_ref, out_ref):
    vid = vid_ref[...]
    val = val_ref[...]
    lane = jax.lax.broadcasted_iota(jnp.int32, (_T, _D), 1)
    oh2 = (vid == lane % _NV).astype(jnp.float32)
    z = oh2 * jnp.where(lane < _NV, val, 1.0)
    hp = jnp.dot(z, P_ref[...], preferred_element_type=jnp.float32)
    h = jnp.maximum(hp, 0.0)
    c = jnp.dot(h, W2c_ref[...], preferred_element_type=jnp.float32)
    vvar = jnp.dot(c * c, Jn_ref[...], preferred_element_type=jnp.float32)
    out_ref[...] = c * jax.lax.rsqrt(vvar + 1e-5) * gam_ref[...] + bet_ref[...]


def _lnt_body(cat_ref, gam_ref, bet_ref, out_ref):
    t = cat_ref[...]
    mean = jnp.mean(t, axis=1, keepdims=True)
    var = jnp.mean((t - mean) ** 2, axis=1, keepdims=True)
    out_ref[...] = (t - mean) / jnp.sqrt(var + 1e-5) * gam_ref[...] + bet_ref[...]


_sc_mesh = plsc.VectorSubcoreMesh(core_axis_name="c", subcore_axis_name="s",
                                  num_cores=_NC, num_subcores=_NS)


@functools.partial(
    pl.kernel,
    out_type=(),
    mesh=_sc_mesh,
    compiler_params=pltpu.CompilerParams(needs_layout_passes=False),
    scratch_types=[
        pltpu.VMEM((_CHTOK,), jnp.int32),
        pltpu.VMEM((_CHTOK,), jnp.int32),
        pltpu.VMEM((_NV,), jnp.int32),
        pltpu.VMEM((_PAD,), jnp.int32),
        pltpu.VMEM((_PAD,), jnp.int32),
        pltpu.VMEM((_MAXCH + 1, _CH), jnp.int32),
        pltpu.VMEM((2, _CH, _D), jnp.float32),
        pltpu.SemaphoreType.DMA,
        pltpu.SemaphoreType.DMA,
    ],
)
def _sc_overwrite(lnt_hbm, vid_hbm, cid_hbm, vty_hbm, out_hbm,
                  vids_v, cids_v, vty_v, srcf, dstf, dst2, rows, gsem, ssem):
    wid = lax.axis_index("s") * _NC + lax.axis_index("c")
    base = wid * _CHTOK
    pltpu.sync_copy(vid_hbm.at[pl.ds(base, _CHTOK)], vids_v)
    pltpu.sync_copy(cid_hbm.at[pl.ds(base, _CHTOK)], cids_v)
    pltpu.sync_copy(vty_hbm, vty_v)

    def step(i, ptr):
        sl = pl.ds(i * _LN, _LN)
        vid = vids_v[sl]
        cid = cids_v[sl]
        vt = plsc.load_gather(vty_v, [vid])
        m = (vt == 1) & (cid >= 0)
        tok = base + i * _LN + lax.iota(jnp.int32, _LN)
        plsc.store_compressed(srcf.at[pl.ds(ptr, _LN)], cid, mask=m)
        plsc.store_compressed(dstf.at[pl.ds(ptr, _LN)], tok, mask=m)
        cnt = plsc.all_reduce_population_count(m)
        return ptr + jnp.max(cnt)

    count = lax.fori_loop(0, _VPT, step, jnp.int32(0))

    @pl.when(count > 0)
    def _():
        lane0 = lax.iota(jnp.int32, _LN) == 0
        s0 = srcf[pl.ds(0, _LN)]
        d0 = dstf[pl.ds(0, _LN)]
        spad = jnp.full((_LN,), jnp.max(jnp.where(lane0, s0, 0)), jnp.int32)
        dpad = jnp.full((_LN,), jnp.max(jnp.where(lane0, d0, 0)), jnp.int32)
        for j in range(_CH // _LN):
            srcf[pl.ds(count + j * _LN, _LN)] = spad
            dstf[pl.ds(count + j * _LN, _LN)] = dpad
        for c in range(_MAXCH + 1):
            for k in range(_CH // _LN):
                dst2[c, pl.ds(k * _LN, _LN)] = dstf[pl.ds(c * _CH + k * _LN, _LN)]

        nch = (count + _CH - 1) // _CH
        pltpu.async_copy(lnt_hbm.at[srcf.at[pl.ds(0, _CH)]], rows.at[0], gsem)

        def chunk(c, carry):
            cur = c & 1
            nxt = (c + 1) & 1
            pltpu.make_async_copy(
                lnt_hbm.at[srcf.at[pl.ds(0, _CH)]], rows.at[cur], gsem).wait()

            @pl.when(c + 1 < nch)
            def _():
                pltpu.async_copy(
                    lnt_hbm.at[srcf.at[pl.ds((c + 1) * _CH, _CH)]],
                    rows.at[nxt], gsem)

            pltpu.async_copy(rows.at[cur], out_hbm.at[dst2.at[c]], ssem)
            pltpu.make_async_copy(rows.at[cur], out_hbm.at[dst2.at[c]],
                                  ssem).wait()
            return carry

        lax.fori_loop(0, nch, chunk, jnp.int32(0))


@jax.jit
def kernel(variate_ids, value_num, cat_ids, variate_type, numeric_means,
           numeric_stds, w1, b1, W2, b2, cat_table, ln_gamma, ln_beta):
    vid2 = variate_ids.reshape(_N, 1).astype(jnp.int32)
    val2 = value_num.reshape(_N, 1)
    tf = variate_type.astype(jnp.int32)

    isnum = (tf == 0).astype(jnp.float32)
    s = 1.0 / (numeric_stds + 1e-6)
    t = -numeric_means * s
    P = jnp.full((_D, _D), 0.0, jnp.float32)
    P = P.at[:_NV, :_H].set(s[:, None] * w1[None, :])
    P = P.at[_NV:, :_H].set(t[:, None] * w1[None, :] + b1[None, :]
                            - 1e20 * (1.0 - isnum)[:, None])
    P = P.at[_NV:, _H].set(isnum)
    P = P.at[_NV:, _H + 1:].set(-1.0)
    W2ext = jnp.zeros((_D, _D), jnp.float32)
    W2ext = W2ext.at[:_H, :].set(W2)
    W2ext = W2ext.at[_H, :].set(b2)
    W2c = W2ext - jnp.mean(W2ext, axis=1, keepdims=True)
    Jn = jnp.full((_D, _D), 1.0 / _D, jnp.float32)

    grid = _N // _T
    tok = lambda i: (i, 0)
    full = lambda i: (0, 0)
    base = pl.pallas_call(
        _tc_base_body,
        grid=(grid,),
        in_specs=[
            pl.BlockSpec((_T, 1), tok),
            pl.BlockSpec((_T, 1), tok),
            pl.BlockSpec((_D, _D), full),
            pl.BlockSpec((_D, _D), full),
            pl.BlockSpec((_D, _D), full),
            pl.BlockSpec((1, _D), full),
            pl.BlockSpec((1, _D), full),
        ],
        out_specs=pl.BlockSpec((_T, _D), tok),
        out_shape=jax.ShapeDtypeStruct((_N, _D), jnp.float32),
    )(vid2, val2, P, W2c, Jn, ln_gamma.reshape(1, _D),
      ln_beta.reshape(1, _D))

    lnt = pl.pallas_call(
        _lnt_body,
        in_specs=[
            pl.BlockSpec((_NCAT, _D), lambda: (0, 0)),
            pl.BlockSpec((1, _D), lambda: (0, 0)),
            pl.BlockSpec((1, _D), lambda: (0, 0)),
        ],
        out_specs=pl.BlockSpec((_NCAT, _D), lambda: (0, 0)),
        out_shape=jax.ShapeDtypeStruct((_NCAT, _D), jnp.float32),
    )(cat_table, ln_gamma.reshape(1, _D), ln_beta.reshape(1, _D))

    out_ref = jax.new_ref(base)
    _sc_overwrite(lnt, variate_ids.reshape(_N).astype(jnp.int32),
                  cat_ids.reshape(_N).astype(jnp.int32), tf, out_ref)
    return jax.freeze(out_ref).reshape(_B, _L, _D)
